# baseline (speedup 1.0000x reference)
import functools

import jax
import jax.numpy as jnp
from jax.experimental import pallas as pl
from jax.experimental.pallas import tpu as pltpu

H = 768
A = 8
N_CODES = 10000
OUT = 512
VERY_NEG = -1e30

TM = 1000
TN = 100
NCHUNK = 5
TB = 64
P4 = OUT // 128
PW = OUT // 256


def _proj_kernel(e_ref, w_ref, b_ref, o_ref):
    eb = e_ref[...].astype(jnp.bfloat16)
    p = jnp.dot(eb, w_ref[...], preferred_element_type=jnp.float32)
    o_ref[...] = (p + b_ref[...]).astype(jnp.bfloat16)


def _project(embed_table, w, b):
    n, d = embed_table.shape[0], w.shape[1]
    return pl.pallas_call(
        _proj_kernel,
        grid=(n // TM,),
        in_specs=[
            pl.BlockSpec((TM, H), lambda i: (i, 0)),
            pl.BlockSpec((H, d), lambda i: (0, 0)),
            pl.BlockSpec((1, d), lambda i: (0, 0)),
        ],
        out_specs=pl.BlockSpec((TM, d), lambda i: (i, 0)),
        out_shape=jax.ShapeDtypeStruct((n, d), jnp.bfloat16),
        compiler_params=pltpu.CompilerParams(
            dimension_semantics=("parallel",),
            vmem_limit_bytes=100 * 1024 * 1024,
        ),
    )(embed_table, w, b)


def _attn_kernel(idx_ref, gl_ref, ga_ref, m3_ref, w2t_ref, ewp_ref,
                 out_ref, scr_ref):
    h = jnp.maximum(gl_ref[...] + ga_ref[...], jnp.bfloat16(0.0))
    lg = jnp.dot(h, w2t_ref[...], preferred_element_type=jnp.float32)
    lg3 = lg.reshape(TN, A, A)
    lg3 = jnp.where(m3_ref[...] > 0.0, lg3, VERY_NEG)
    mx = jnp.max(lg3, axis=1, keepdims=True)
    e3 = jnp.exp(lg3 - mx)
    attn3 = e3 / jnp.sum(e3, axis=1, keepdims=True)

    def row(c, _):
        for a in range(A):
            i2 = pl.multiple_of(idx_ref[0, 0, c * A + a], PW)
            scr_ref[a * TN + c] = ewp_ref[pl.ds(i2, PW), :]
        return ()

    jax.lax.fori_loop(0, TN, row, ())
    out_ref[...] = functools.reduce(
        lambda x, y: x + y,
        [
            pltpu.bitcast(scr_ref[a * TN:(a + 1) * TN], jnp.bfloat16)
            .astype(jnp.float32) * attn3[:, a:a + 1, :1]
            for a in range(A)
        ],
    )


def _attention(anc2, gl, ga, mask3, w2t, ewp, c0, nc):
    return pl.pallas_call(
        _attn_kernel,
        grid=(nc // TN,),
        in_specs=[
            pl.BlockSpec((1, 1, A * TN), lambda i: (c0 + i, 0, 0),
                         memory_space=pltpu.SMEM),
            pl.BlockSpec((A * TN, H), lambda i: (c0 + i, 0)),
            pl.BlockSpec((A * TN, H), lambda i: (i, 0)),
            pl.BlockSpec((TN, A, A), lambda i: (c0 + i, 0, 0)),
            pl.BlockSpec((H, A), lambda i: (0, 0)),
            pl.BlockSpec((N_CODES * 4 * PW, 128), lambda i: (0, 0)),
        ],
        out_specs=pl.BlockSpec((TN, P4, 128), lambda i: (i, 0, 0)),
        out_shape=jax.ShapeDtypeStruct((nc, P4, 128), jnp.float32),
        scratch_shapes=[pltpu.VMEM((A * TN, PW, 128), jnp.int32)],
        compiler_params=pltpu.CompilerParams(
            dimension_semantics=("parallel",),
            vmem_limit_bytes=60 * 1024 * 1024,
        ),
    )(anc2, gl, ga, mask3, w2t, ewp)


def _pool_kernel(idx_ref, wgt_ref, cm_ref, c3_ref, bc_ref, out_ref):
    def row(r, _):
        acc = jnp.zeros((P4, 128), jnp.float32)
        cnt = jnp.float32(0.0)
        for m in range(48):
            acc = acc + wgt_ref[r, m] * c3_ref[idx_ref[r, m]]
            cnt = cnt + cm_ref[r, m]
        scale = 1.0 / jnp.maximum(jnp.full((P4, 128), cnt), 1.0)
        out_ref[r] = acc * scale + bc_ref[...]
        return ()

    jax.lax.fori_loop(0, TB, row, ())


def _pool(idx, wgt, cmf, c3, bc4):
    bv = idx.shape[0]
    return pl.pallas_call(
        _pool_kernel,
        grid=(bv // TB,),
        in_specs=[
            pl.BlockSpec((TB, 48), lambda i: (i, 0),
                         memory_space=pltpu.SMEM),
            pl.BlockSpec((TB, 48), lambda i: (i, 0),
                         memory_space=pltpu.SMEM),
            pl.BlockSpec((TB, 48), lambda i: (i, 0),
                         memory_space=pltpu.SMEM),
            pl.BlockSpec((N_CODES, P4, 128), lambda i: (0, 0, 0)),
            pl.BlockSpec((P4, 128), lambda i: (0, 0)),
        ],
        out_specs=pl.BlockSpec((TB, P4, 128), lambda i: (i, 0, 0)),
        out_shape=jax.ShapeDtypeStruct((bv, P4, 128), jnp.float32),
        compiler_params=pltpu.CompilerParams(
            dimension_semantics=("parallel",),
            vmem_limit_bytes=60 * 1024 * 1024,
        ),
    )(idx, wgt, cmf, c3, bc4)


def kernel(embed_table, w1, b1, w2, b2, wc, bc, masks, code_mask,
           leaves_list, ancestors_list, input_ids):
    del b2
    f32 = jnp.float32
    w_l = w1[:H, :].astype(jnp.bfloat16)
    w_ae = jnp.concatenate([w1[H:, :], wc], axis=1).astype(jnp.bfloat16)
    b_l = jnp.zeros((1, H), f32)
    b_ae = jnp.concatenate([b1, jnp.zeros((OUT,), f32)]).reshape(1, H + OUT)
    ql = _project(embed_table, w_l, b_l)
    qae = _project(embed_table, w_ae, b_ae)
    qa, ew = qae[:, :H], qae[:, H:]
    ewp = jax.lax.bitcast_convert_type(
        ew.reshape(N_CODES * 4, PW, 2, 128).transpose(0, 1, 3, 2),
        jnp.int32).reshape(N_CODES * 4 * PW, 128)

    gl = ql[leaves_list.reshape(-1).astype(jnp.int32)]
    anc_flat = ancestors_list.reshape(-1).astype(jnp.int32)
    anc2 = (anc_flat * PW).reshape(-1, 1, A * TN)
    mask3 = jnp.broadcast_to(masks[:, :, None], (N_CODES, A, A))
    w2t = jnp.broadcast_to(w2, (H, A)).astype(jnp.bfloat16)
    nc = N_CODES // NCHUNK
    chunks = []
    for c in range(NCHUNK):
        ga = qa[anc_flat[c * nc * A:(c + 1) * nc * A]]
        chunks.append(
            _attention(anc2, gl, ga, mask3, w2t, ewp, c * (nc // TN), nc))
    ctab = jnp.concatenate(chunks, axis=0)

    ids = input_ids.reshape(-1, 48).astype(jnp.int32)
    idx = jnp.maximum(ids - 1, 0)
    cmf = code_mask.reshape(-1, 48).astype(f32)
    wgt = cmf * (ids != 0).astype(f32)
    bc4 = bc.reshape(P4, 128)
    out = _pool(idx, wgt, cmf, ctab, bc4)
    B, V, _ = input_ids.shape
    return out.reshape(B, V, OUT)


# speedup vs baseline: 1.2616x; 1.2616x over previous
import functools

import jax
import jax.numpy as jnp
from jax.experimental import pallas as pl
from jax.experimental.pallas import tpu as pltpu

H = 768
A = 8
N_CODES = 10000
OUT = 512
VERY_NEG = -1e30

TM = 1000
TN = 100
NCHUNK = 5
TB = 64
P4 = OUT // 128
PW = OUT // 256


def _proj_kernel(e_ref, w_ref, b_ref, o_ref):
    eb = e_ref[...].astype(jnp.bfloat16)
    p = jnp.dot(eb, w_ref[...], preferred_element_type=jnp.float32)
    o_ref[...] = (p + b_ref[...]).astype(jnp.bfloat16)


def _projb_kernel(e_ref, w_ref, b_ref, qa_ref, ewp_ref):
    eb = e_ref[...].astype(jnp.bfloat16)
    p = jnp.dot(eb, w_ref[...], preferred_element_type=jnp.float32)
    p = p + b_ref[...]
    qa_ref[...] = p[:, :H].astype(jnp.bfloat16)
    pcs = []
    for r in range(PW):
        lo = p[:, H + r * 256:H + r * 256 + 128]
        hi = p[:, H + r * 256 + 128:H + r * 256 + 256]
        lob = pltpu.bitcast(
            lo.astype(jnp.bfloat16).astype(jnp.float32), jnp.uint32)
        hib = pltpu.bitcast(
            hi.astype(jnp.bfloat16).astype(jnp.float32), jnp.uint32)
        packed = (lob >> jnp.uint32(16)) | (hib & jnp.uint32(0xFFFF0000))
        pcs.append(pltpu.bitcast(packed, jnp.int32).reshape(TM, 1, 128))
    ewp_ref[...] = jnp.concatenate(pcs, axis=1)


def _projb(embed_table, w, b):
    n, d = embed_table.shape[0], w.shape[1]
    return pl.pallas_call(
        _projb_kernel,
        grid=(n // TM,),
        in_specs=[
            pl.BlockSpec((TM, H), lambda i: (i, 0)),
            pl.BlockSpec((H, d), lambda i: (0, 0)),
            pl.BlockSpec((1, d), lambda i: (0, 0)),
        ],
        out_specs=[
            pl.BlockSpec((TM, H), lambda i: (i, 0)),
            pl.BlockSpec((TM, PW, 128), lambda i: (i, 0, 0)),
        ],
        out_shape=[
            jax.ShapeDtypeStruct((n, H), jnp.bfloat16),
            jax.ShapeDtypeStruct((n, PW, 128), jnp.int32),
        ],
        compiler_params=pltpu.CompilerParams(
            dimension_semantics=("parallel",),
            vmem_limit_bytes=100 * 1024 * 1024,
        ),
    )(embed_table, w, b)


def _project(embed_table, w, b):
    n, d = embed_table.shape[0], w.shape[1]
    return pl.pallas_call(
        _proj_kernel,
        grid=(n // TM,),
        in_specs=[
            pl.BlockSpec((TM, H), lambda i: (i, 0)),
            pl.BlockSpec((H, d), lambda i: (0, 0)),
            pl.BlockSpec((1, d), lambda i: (0, 0)),
        ],
        out_specs=pl.BlockSpec((TM, d), lambda i: (i, 0)),
        out_shape=jax.ShapeDtypeStruct((n, d), jnp.bfloat16),
        compiler_params=pltpu.CompilerParams(
            dimension_semantics=("parallel",),
            vmem_limit_bytes=100 * 1024 * 1024,
        ),
    )(embed_table, w, b)


def _attn_kernel(idx_ref, gl_ref, ga_ref, m3_ref, w2t_ref, ewp_ref,
                 out_ref, scr_ref):
    h = jnp.maximum(gl_ref[...] + ga_ref[...], jnp.bfloat16(0.0))
    lg = jnp.dot(h, w2t_ref[...], preferred_element_type=jnp.float32)
    lg3 = lg.reshape(TN, A, A)
    lg3 = jnp.where(m3_ref[...] > 0.0, lg3, VERY_NEG)
    mx = jnp.max(lg3, axis=1, keepdims=True)
    e3 = jnp.exp(lg3 - mx)
    attn3 = e3 / jnp.sum(e3, axis=1, keepdims=True)

    def row(c, _):
        for a in range(A):
            scr_ref[a * TN + c] = ewp_ref[idx_ref[0, 0, c * A + a]]
        return ()

    jax.lax.fori_loop(0, TN, row, ())
    out_ref[...] = functools.reduce(
        lambda x, y: x + y,
        [
            pltpu.bitcast(scr_ref[a * TN:(a + 1) * TN], jnp.bfloat16)
            .astype(jnp.float32) * attn3[:, a:a + 1, :1]
            for a in range(A)
        ],
    )


def _attention(anc2, gl, ga, mask3, w2t, ewp, c0, nc):
    return pl.pallas_call(
        _attn_kernel,
        grid=(nc // TN,),
        in_specs=[
            pl.BlockSpec((1, 1, A * TN), lambda i: (c0 + i, 0, 0),
                         memory_space=pltpu.SMEM),
            pl.BlockSpec((A * TN, H), lambda i: (c0 + i, 0)),
            pl.BlockSpec((A * TN, H), lambda i: (i, 0)),
            pl.BlockSpec((TN, A, A), lambda i: (c0 + i, 0, 0)),
            pl.BlockSpec((H, A), lambda i: (0, 0)),
            pl.BlockSpec((N_CODES * 4, PW, 128), lambda i: (0, 0, 0)),
        ],
        out_specs=pl.BlockSpec((TN, P4, 128), lambda i: (i, 0, 0)),
        out_shape=jax.ShapeDtypeStruct((nc, P4, 128), jnp.float32),
        scratch_shapes=[pltpu.VMEM((A * TN, PW, 128), jnp.int32)],
        compiler_params=pltpu.CompilerParams(
            dimension_semantics=("parallel",),
            vmem_limit_bytes=60 * 1024 * 1024,
        ),
    )(anc2, gl, ga, mask3, w2t, ewp)


def _pool_kernel(idx_ref, wgt_ref, cm_ref, c3_ref, bc_ref, out_ref):
    def row(r, _):
        acc = jnp.zeros((P4, 128), jnp.float32)
        cnt = jnp.float32(0.0)
        for m in range(48):
            acc = acc + wgt_ref[r, m] * c3_ref[idx_ref[r, m]]
            cnt = cnt + cm_ref[r, m]
        scale = 1.0 / jnp.maximum(jnp.full((P4, 128), cnt), 1.0)
        out_ref[r] = acc * scale + bc_ref[...]
        return ()

    jax.lax.fori_loop(0, TB, row, ())


def _pool(idx, wgt, cmf, c3, bc4):
    bv = idx.shape[0]
    return pl.pallas_call(
        _pool_kernel,
        grid=(bv // TB,),
        in_specs=[
            pl.BlockSpec((TB, 48), lambda i: (i, 0),
                         memory_space=pltpu.SMEM),
            pl.BlockSpec((TB, 48), lambda i: (i, 0),
                         memory_space=pltpu.SMEM),
            pl.BlockSpec((TB, 48), lambda i: (i, 0),
                         memory_space=pltpu.SMEM),
            pl.BlockSpec((N_CODES, P4, 128), lambda i: (0, 0, 0)),
            pl.BlockSpec((P4, 128), lambda i: (0, 0)),
        ],
        out_specs=pl.BlockSpec((TB, P4, 128), lambda i: (i, 0, 0)),
        out_shape=jax.ShapeDtypeStruct((bv, P4, 128), jnp.float32),
        compiler_params=pltpu.CompilerParams(
            dimension_semantics=("parallel",),
            vmem_limit_bytes=60 * 1024 * 1024,
        ),
    )(idx, wgt, cmf, c3, bc4)


def kernel(embed_table, w1, b1, w2, b2, wc, bc, masks, code_mask,
           leaves_list, ancestors_list, input_ids):
    del b2
    f32 = jnp.float32
    w_l = w1[:H, :].astype(jnp.bfloat16)
    w_ae = jnp.concatenate([w1[H:, :], wc], axis=1).astype(jnp.bfloat16)
    b_l = jnp.zeros((1, H), f32)
    b_ae = jnp.concatenate([b1, jnp.zeros((OUT,), f32)]).reshape(1, H + OUT)
    ql = _project(embed_table, w_l, b_l)
    qa, ewp = _projb(embed_table, w_ae, b_ae)

    gl = ql[leaves_list.reshape(-1).astype(jnp.int32)]
    anc_flat = ancestors_list.reshape(-1).astype(jnp.int32)
    anc2 = anc_flat.reshape(-1, 1, A * TN)
    mask3 = jnp.broadcast_to(masks[:, :, None], (N_CODES, A, A))
    w2t = jnp.broadcast_to(w2, (H, A)).astype(jnp.bfloat16)
    nc = N_CODES // NCHUNK
    chunks = []
    for c in range(NCHUNK):
        ga = qa[anc_flat[c * nc * A:(c + 1) * nc * A]]
        chunks.append(
            _attention(anc2, gl, ga, mask3, w2t, ewp, c * (nc // TN), nc))
    ctab = jnp.concatenate(chunks, axis=0)

    ids = input_ids.reshape(-1, 48).astype(jnp.int32)
    idx = jnp.maximum(ids - 1, 0)
    cmf = code_mask.reshape(-1, 48).astype(f32)
    wgt = cmf * (ids != 0).astype(f32)
    bc4 = bc.reshape(P4, 128)
    out = _pool(idx, wgt, cmf, ctab, bc4)
    B, V, _ = input_ids.shape
    return out.reshape(B, V, OUT)


# speedup vs baseline: 1.3269x; 1.0517x over previous
import functools

import jax
import jax.numpy as jnp
from jax.experimental import pallas as pl
from jax.experimental.pallas import tpu as pltpu

H = 768
A = 8
N_CODES = 10000
OUT = 512
VERY_NEG = -1e30

TM = 1000
TN = 100
NCHUNK = 5
TB = 64
P4 = OUT // 128
PW = OUT // 256


def _proj_kernel(e_ref, w_ref, b_ref, o_ref):
    eb = e_ref[...].astype(jnp.bfloat16)
    p = jnp.dot(eb, w_ref[...], preferred_element_type=jnp.float32)
    o_ref[...] = (p + b_ref[...]).astype(jnp.bfloat16)


def _projb_kernel(e_ref, w_ref, b_ref, qa_ref, ewp_ref):
    eb = e_ref[...].astype(jnp.bfloat16)
    p = jnp.dot(eb, w_ref[...], preferred_element_type=jnp.float32)
    p = p + b_ref[...]
    qa_ref[...] = p[:, :H].astype(jnp.bfloat16)
    pcs = []
    for r in range(PW):
        lo = p[:, H + r * 256:H + r * 256 + 128]
        hi = p[:, H + r * 256 + 128:H + r * 256 + 256]
        lob = pltpu.bitcast(
            lo.astype(jnp.bfloat16).astype(jnp.float32), jnp.uint32)
        hib = pltpu.bitcast(
            hi.astype(jnp.bfloat16).astype(jnp.float32), jnp.uint32)
        packed = (lob >> jnp.uint32(16)) | (hib & jnp.uint32(0xFFFF0000))
        pcs.append(pltpu.bitcast(packed, jnp.int32).reshape(TM, 1, 128))
    ewp_ref[...] = jnp.concatenate(pcs, axis=1)


def _projb(embed_table, w, b):
    n, d = embed_table.shape[0], w.shape[1]
    return pl.pallas_call(
        _projb_kernel,
        grid=(n // TM,),
        in_specs=[
            pl.BlockSpec((TM, H), lambda i: (i, 0)),
            pl.BlockSpec((H, d), lambda i: (0, 0)),
            pl.BlockSpec((1, d), lambda i: (0, 0)),
        ],
        out_specs=[
            pl.BlockSpec((TM, H), lambda i: (i, 0)),
            pl.BlockSpec((TM, PW, 128), lambda i: (i, 0, 0)),
        ],
        out_shape=[
            jax.ShapeDtypeStruct((n, H), jnp.bfloat16),
            jax.ShapeDtypeStruct((n, PW, 128), jnp.int32),
        ],
        compiler_params=pltpu.CompilerParams(
            dimension_semantics=("parallel",),
            vmem_limit_bytes=100 * 1024 * 1024,
        ),
    )(embed_table, w, b)


def _project(embed_table, w, b):
    n, d = embed_table.shape[0], w.shape[1]
    return pl.pallas_call(
        _proj_kernel,
        grid=(n // TM,),
        in_specs=[
            pl.BlockSpec((TM, H), lambda i: (i, 0)),
            pl.BlockSpec((H, d), lambda i: (0, 0)),
            pl.BlockSpec((1, d), lambda i: (0, 0)),
        ],
        out_specs=pl.BlockSpec((TM, d), lambda i: (i, 0)),
        out_shape=jax.ShapeDtypeStruct((n, d), jnp.bfloat16),
        compiler_params=pltpu.CompilerParams(
            dimension_semantics=("parallel",),
            vmem_limit_bytes=100 * 1024 * 1024,
        ),
    )(embed_table, w, b)


def _attn_kernel(idx_ref, gl_ref, ga_ref, m_ref, w2t_ref, ewp_ref,
                 out_ref, scr_ref):
    h = jnp.maximum(gl_ref[...] + ga_ref[...], jnp.bfloat16(0.0))
    lg = jnp.dot(h, w2t_ref[...], preferred_element_type=jnp.float32)
    lg = lg + jnp.broadcast_to(m_ref[...], (A * TN, A))
    lg3 = lg.reshape(TN, A, A)
    mx = jnp.max(lg3, axis=1, keepdims=True)
    e3 = jnp.exp(lg3 - mx)
    attn3 = e3 / jnp.sum(e3, axis=1, keepdims=True)

    def row(c4, _):
        c = c4 * 4
        for u in range(4):
            for a in range(A):
                scr_ref[a * TN + c + u] = ewp_ref[idx_ref[0, 0,
                                                          (c + u) * A + a]]
        return ()

    jax.lax.fori_loop(0, TN // 4, row, ())
    out_ref[...] = functools.reduce(
        lambda x, y: x + y,
        [
            pltpu.bitcast(scr_ref[a * TN:(a + 1) * TN], jnp.bfloat16)
            .astype(jnp.float32) * attn3[:, a:a + 1, :1]
            for a in range(A)
        ],
    )


def _attention(anc2, gl, ga, mask3, w2t, ewp, c0, nc):
    return pl.pallas_call(
        _attn_kernel,
        grid=(nc // TN,),
        in_specs=[
            pl.BlockSpec((1, 1, A * TN), lambda i: (c0 + i, 0, 0),
                         memory_space=pltpu.SMEM),
            pl.BlockSpec((A * TN, H), lambda i: (c0 + i, 0)),
            pl.BlockSpec((A * TN, H), lambda i: (i, 0)),
            pl.BlockSpec((A * TN, 1), lambda i: (c0 + i, 0)),
            pl.BlockSpec((H, A), lambda i: (0, 0)),
            pl.BlockSpec((N_CODES * 4, PW, 128), lambda i: (0, 0, 0)),
        ],
        out_specs=pl.BlockSpec((TN, P4, 128), lambda i: (i, 0, 0)),
        out_shape=jax.ShapeDtypeStruct((nc, P4, 128), jnp.float32),
        scratch_shapes=[pltpu.VMEM((A * TN, PW, 128), jnp.int32)],
        compiler_params=pltpu.CompilerParams(
            dimension_semantics=("parallel",),
            vmem_limit_bytes=60 * 1024 * 1024,
        ),
    )(anc2, gl, ga, mask3, w2t, ewp)


def _pool_kernel(idx_ref, wgt_ref, cm_ref, c3_ref, bc_ref, out_ref):
    def row(r, _):
        acc = jnp.zeros((P4, 128), jnp.float32)
        cnt = jnp.float32(0.0)
        for m in range(48):
            acc = acc + wgt_ref[r, m] * c3_ref[idx_ref[r, m]]
            cnt = cnt + cm_ref[r, m]
        scale = 1.0 / jnp.maximum(jnp.full((P4, 128), cnt), 1.0)
        out_ref[r] = acc * scale + bc_ref[...]
        return ()

    jax.lax.fori_loop(0, TB, row, ())


def _pool(idx, wgt, cmf, c3, bc4):
    bv = idx.shape[0]
    return pl.pallas_call(
        _pool_kernel,
        grid=(bv // TB,),
        in_specs=[
            pl.BlockSpec((TB, 48), lambda i: (i, 0),
                         memory_space=pltpu.SMEM),
            pl.BlockSpec((TB, 48), lambda i: (i, 0),
                         memory_space=pltpu.SMEM),
            pl.BlockSpec((TB, 48), lambda i: (i, 0),
                         memory_space=pltpu.SMEM),
            pl.BlockSpec((N_CODES, P4, 128), lambda i: (0, 0, 0)),
            pl.BlockSpec((P4, 128), lambda i: (0, 0)),
        ],
        out_specs=pl.BlockSpec((TB, P4, 128), lambda i: (i, 0, 0)),
        out_shape=jax.ShapeDtypeStruct((bv, P4, 128), jnp.float32),
        compiler_params=pltpu.CompilerParams(
            dimension_semantics=("parallel",),
            vmem_limit_bytes=60 * 1024 * 1024,
        ),
    )(idx, wgt, cmf, c3, bc4)


def kernel(embed_table, w1, b1, w2, b2, wc, bc, masks, code_mask,
           leaves_list, ancestors_list, input_ids):
    del b2
    f32 = jnp.float32
    w_l = w1[:H, :].astype(jnp.bfloat16)
    w_ae = jnp.concatenate([w1[H:, :], wc], axis=1).astype(jnp.bfloat16)
    b_l = jnp.zeros((1, H), f32)
    b_ae = jnp.concatenate([b1, jnp.zeros((OUT,), f32)]).reshape(1, H + OUT)
    ql = _project(embed_table, w_l, b_l)
    qa, ewp = _projb(embed_table, w_ae, b_ae)

    gl = ql[leaves_list.reshape(-1).astype(jnp.int32)]
    anc_flat = ancestors_list.reshape(-1).astype(jnp.int32)
    anc2 = anc_flat.reshape(-1, 1, A * TN)
    madd = ((1.0 - masks) * VERY_NEG).reshape(N_CODES * A, 1)
    w2t = jnp.broadcast_to(w2, (H, A)).astype(jnp.bfloat16)
    nc = N_CODES // NCHUNK
    chunks = []
    for c in range(NCHUNK):
        ga = qa[anc_flat[c * nc * A:(c + 1) * nc * A]]
        chunks.append(
            _attention(anc2, gl, ga, madd, w2t, ewp, c * (nc // TN), nc))
    ctab = jnp.concatenate(chunks, axis=0)

    ids = input_ids.reshape(-1, 48).astype(jnp.int32)
    idx = jnp.maximum(ids - 1, 0)
    cmf = code_mask.reshape(-1, 48).astype(f32)
    wgt = cmf * (ids != 0).astype(f32)
    bc4 = bc.reshape(P4, 128)
    out = _pool(idx, wgt, cmf, ctab, bc4)
    B, V, _ = input_ids.shape
    return out.reshape(B, V, OUT)


# speedup vs baseline: 1.3473x; 1.0154x over previous
import functools

import jax
import jax.numpy as jnp
from jax.experimental import pallas as pl
from jax.experimental.pallas import tpu as pltpu

H = 768
A = 8
N_CODES = 10000
OUT = 512
VERY_NEG = -1e30

TM = 1000
TN = 200
NCHUNK = 5
TB = 64
P4 = OUT // 128
PW = OUT // 256


def _proj_kernel(e_ref, w_ref, b_ref, o_ref):
    eb = e_ref[...].astype(jnp.bfloat16)
    p = jnp.dot(eb, w_ref[...], preferred_element_type=jnp.float32)
    o_ref[...] = (p + b_ref[...]).astype(jnp.bfloat16)


def _projb_kernel(e_ref, w_ref, b_ref, qa_ref, ewp_ref):
    eb = e_ref[...].astype(jnp.bfloat16)
    p = jnp.dot(eb, w_ref[...], preferred_element_type=jnp.float32)
    p = p + b_ref[...]
    qa_ref[...] = p[:, :H].astype(jnp.bfloat16)
    pcs = []
    for r in range(PW):
        lo = p[:, H + r * 256:H + r * 256 + 128]
        hi = p[:, H + r * 256 + 128:H + r * 256 + 256]
        lob = pltpu.bitcast(
            lo.astype(jnp.bfloat16).astype(jnp.float32), jnp.uint32)
        hib = pltpu.bitcast(
            hi.astype(jnp.bfloat16).astype(jnp.float32), jnp.uint32)
        packed = (lob >> jnp.uint32(16)) | (hib & jnp.uint32(0xFFFF0000))
        pcs.append(pltpu.bitcast(packed, jnp.int32).reshape(TM, 1, 128))
    ewp_ref[...] = jnp.concatenate(pcs, axis=1)


def _projb(embed_table, w, b):
    n, d = embed_table.shape[0], w.shape[1]
    return pl.pallas_call(
        _projb_kernel,
        grid=(n // TM,),
        in_specs=[
            pl.BlockSpec((TM, H), lambda i: (i, 0)),
            pl.BlockSpec((H, d), lambda i: (0, 0)),
            pl.BlockSpec((1, d), lambda i: (0, 0)),
        ],
        out_specs=[
            pl.BlockSpec((TM, H), lambda i: (i, 0)),
            pl.BlockSpec((TM, PW, 128), lambda i: (i, 0, 0)),
        ],
        out_shape=[
            jax.ShapeDtypeStruct((n, H), jnp.bfloat16),
            jax.ShapeDtypeStruct((n, PW, 128), jnp.int32),
        ],
        compiler_params=pltpu.CompilerParams(
            dimension_semantics=("parallel",),
            vmem_limit_bytes=100 * 1024 * 1024,
        ),
    )(embed_table, w, b)


def _project(embed_table, w, b):
    n, d = embed_table.shape[0], w.shape[1]
    return pl.pallas_call(
        _proj_kernel,
        grid=(n // TM,),
        in_specs=[
            pl.BlockSpec((TM, H), lambda i: (i, 0)),
            pl.BlockSpec((H, d), lambda i: (0, 0)),
            pl.BlockSpec((1, d), lambda i: (0, 0)),
        ],
        out_specs=pl.BlockSpec((TM, d), lambda i: (i, 0)),
        out_shape=jax.ShapeDtypeStruct((n, d), jnp.bfloat16),
        compiler_params=pltpu.CompilerParams(
            dimension_semantics=("parallel",),
            vmem_limit_bytes=100 * 1024 * 1024,
        ),
    )(embed_table, w, b)


def _attn_kernel(idx_ref, gl_ref, ga_ref, m_ref, w2t_ref, ewp_ref,
                 out_ref, scr_ref):
    h = jnp.maximum(gl_ref[...] + ga_ref[...], jnp.bfloat16(0.0))
    lg = jnp.dot(h, w2t_ref[...], preferred_element_type=jnp.float32)
    lg = lg + jnp.broadcast_to(m_ref[...], (A * TN, A))
    lg3 = lg.reshape(TN, A, A)
    mx = jnp.max(lg3, axis=1, keepdims=True)
    e3 = jnp.exp(lg3 - mx)
    attn3 = e3 / jnp.sum(e3, axis=1, keepdims=True)

    def row(c4, _):
        c = c4 * 4
        for u in range(4):
            for a in range(A):
                scr_ref[a * TN + c + u] = ewp_ref[idx_ref[0, 0,
                                                          (c + u) * A + a]]
        return ()

    jax.lax.fori_loop(0, TN // 4, row, ())
    out_ref[...] = functools.reduce(
        lambda x, y: x + y,
        [
            pltpu.bitcast(scr_ref[a * TN:(a + 1) * TN], jnp.bfloat16)
            .astype(jnp.float32) * attn3[:, a:a + 1, :1]
            for a in range(A)
        ],
    )


def _attention(anc2, gl, ga, mask3, w2t, ewp, c0, nc):
    return pl.pallas_call(
        _attn_kernel,
        grid=(nc // TN,),
        in_specs=[
            pl.BlockSpec((1, 1, A * TN), lambda i: (c0 + i, 0, 0),
                         memory_space=pltpu.SMEM),
            pl.BlockSpec((A * TN, H), lambda i: (c0 + i, 0)),
            pl.BlockSpec((A * TN, H), lambda i: (i, 0)),
            pl.BlockSpec((A * TN, 1), lambda i: (c0 + i, 0)),
            pl.BlockSpec((H, A), lambda i: (0, 0)),
            pl.BlockSpec((N_CODES * 4, PW, 128), lambda i: (0, 0, 0)),
        ],
        out_specs=pl.BlockSpec((TN, P4, 128), lambda i: (i, 0, 0)),
        out_shape=jax.ShapeDtypeStruct((nc, P4, 128), jnp.float32),
        scratch_shapes=[pltpu.VMEM((A * TN, PW, 128), jnp.int32)],
        compiler_params=pltpu.CompilerParams(
            dimension_semantics=("parallel",),
            vmem_limit_bytes=60 * 1024 * 1024,
        ),
    )(anc2, gl, ga, mask3, w2t, ewp)


def _pool_kernel(idx_ref, wgt_ref, cm_ref, c3_ref, bc_ref, out_ref):
    def row(r, _):
        acc = jnp.zeros((P4, 128), jnp.float32)
        cnt = jnp.float32(0.0)
        for m in range(48):
            acc = acc + wgt_ref[r, m] * c3_ref[idx_ref[r, m]]
            cnt = cnt + cm_ref[r, m]
        scale = 1.0 / jnp.maximum(jnp.full((P4, 128), cnt), 1.0)
        out_ref[r] = acc * scale + bc_ref[...]
        return ()

    jax.lax.fori_loop(0, TB, row, ())


def _pool(idx, wgt, cmf, c3, bc4):
    bv = idx.shape[0]
    return pl.pallas_call(
        _pool_kernel,
        grid=(bv // TB,),
        in_specs=[
            pl.BlockSpec((TB, 48), lambda i: (i, 0),
                         memory_space=pltpu.SMEM),
            pl.BlockSpec((TB, 48), lambda i: (i, 0),
                         memory_space=pltpu.SMEM),
            pl.BlockSpec((TB, 48), lambda i: (i, 0),
                         memory_space=pltpu.SMEM),
            pl.BlockSpec((N_CODES, P4, 128), lambda i: (0, 0, 0)),
            pl.BlockSpec((P4, 128), lambda i: (0, 0)),
        ],
        out_specs=pl.BlockSpec((TB, P4, 128), lambda i: (i, 0, 0)),
        out_shape=jax.ShapeDtypeStruct((bv, P4, 128), jnp.float32),
        compiler_params=pltpu.CompilerParams(
            dimension_semantics=("parallel",),
            vmem_limit_bytes=60 * 1024 * 1024,
        ),
    )(idx, wgt, cmf, c3, bc4)


def kernel(embed_table, w1, b1, w2, b2, wc, bc, masks, code_mask,
           leaves_list, ancestors_list, input_ids):
    del b2
    f32 = jnp.float32
    w_l = w1[:H, :].astype(jnp.bfloat16)
    w_ae = jnp.concatenate([w1[H:, :], wc], axis=1).astype(jnp.bfloat16)
    b_l = jnp.zeros((1, H), f32)
    b_ae = jnp.concatenate([b1, jnp.zeros((OUT,), f32)]).reshape(1, H + OUT)
    ql = _project(embed_table, w_l, b_l)
    qa, ewp = _projb(embed_table, w_ae, b_ae)

    gl = ql[leaves_list.reshape(-1).astype(jnp.int32)]
    anc_flat = ancestors_list.reshape(-1).astype(jnp.int32)
    anc2 = anc_flat.reshape(-1, 1, A * TN)
    madd = ((1.0 - masks) * VERY_NEG).reshape(N_CODES * A, 1)
    w2t = jnp.broadcast_to(w2, (H, A)).astype(jnp.bfloat16)
    nc = N_CODES // NCHUNK
    chunks = []
    for c in range(NCHUNK):
        ga = qa[anc_flat[c * nc * A:(c + 1) * nc * A]]
        chunks.append(
            _attention(anc2, gl, ga, madd, w2t, ewp, c * (nc // TN), nc))
    ctab = jnp.concatenate(chunks, axis=0)

    ids = input_ids.reshape(-1, 48).astype(jnp.int32)
    idx = jnp.maximum(ids - 1, 0)
    cmf = code_mask.reshape(-1, 48).astype(f32)
    wgt = cmf * (ids != 0).astype(f32)
    bc4 = bc.reshape(P4, 128)
    out = _pool(idx, wgt, cmf, ctab, bc4)
    B, V, _ = input_ids.shape
    return out.reshape(B, V, OUT)


# speedup vs baseline: 1.3870x; 1.0294x over previous
import functools

import jax
import jax.numpy as jnp
from jax.experimental import pallas as pl
from jax.experimental.pallas import tpu as pltpu

H = 768
A = 8
N_CODES = 10000
OUT = 512
VERY_NEG = -1e30

TM = 1000
TN = 200
NCHUNK = 5
TB = 64
P4 = OUT // 128
PW = OUT // 256


def _proj_kernel(e_ref, w_ref, b_ref, ql_ref, qa_ref, ewp_ref):
    eb = e_ref[...].astype(jnp.bfloat16)
    p = jnp.dot(eb, w_ref[...], preferred_element_type=jnp.float32)
    p = p + b_ref[...]
    ql_ref[...] = p[:, :H].astype(jnp.bfloat16)
    qa_ref[...] = p[:, H:2 * H].astype(jnp.bfloat16)
    pcs = []
    for r in range(PW):
        lo = p[:, 2 * H + r * 256:2 * H + r * 256 + 128]
        hi = p[:, 2 * H + r * 256 + 128:2 * H + r * 256 + 256]
        lob = pltpu.bitcast(
            lo.astype(jnp.bfloat16).astype(jnp.float32), jnp.uint32)
        hib = pltpu.bitcast(
            hi.astype(jnp.bfloat16).astype(jnp.float32), jnp.uint32)
        packed = (lob >> jnp.uint32(16)) | (hib & jnp.uint32(0xFFFF0000))
        pcs.append(pltpu.bitcast(packed, jnp.int32).reshape(TM, 1, 128))
    ewp_ref[...] = jnp.concatenate(pcs, axis=1)


def _project(embed_table, w, b):
    n, d = embed_table.shape[0], w.shape[1]
    return pl.pallas_call(
        _proj_kernel,
        grid=(n // TM,),
        in_specs=[
            pl.BlockSpec((TM, H), lambda i: (i, 0)),
            pl.BlockSpec((H, d), lambda i: (0, 0)),
            pl.BlockSpec((1, d), lambda i: (0, 0)),
        ],
        out_specs=[
            pl.BlockSpec((TM, H), lambda i: (i, 0)),
            pl.BlockSpec((TM, H), lambda i: (i, 0)),
            pl.BlockSpec((TM, PW, 128), lambda i: (i, 0, 0)),
        ],
        out_shape=[
            jax.ShapeDtypeStruct((n, H), jnp.bfloat16),
            jax.ShapeDtypeStruct((n, H), jnp.bfloat16),
            jax.ShapeDtypeStruct((n, PW, 128), jnp.int32),
        ],
        compiler_params=pltpu.CompilerParams(
            dimension_semantics=("parallel",),
            vmem_limit_bytes=100 * 1024 * 1024,
        ),
    )(embed_table, w, b)


def _attn_kernel(idx_ref, gl_ref, ga_ref, m_ref, w2t_ref, ewp_ref,
                 out_ref, scr_ref):
    h = jnp.maximum(gl_ref[...] + ga_ref[...], jnp.bfloat16(0.0))
    lg = jnp.dot(h, w2t_ref[...], preferred_element_type=jnp.float32)
    lg = lg + jnp.broadcast_to(m_ref[...], (A * TN, A))
    lg3 = lg.reshape(TN, A, A)
    mx = jnp.max(lg3, axis=1, keepdims=True)
    e3 = jnp.exp(lg3 - mx)
    attn3 = e3 / jnp.sum(e3, axis=1, keepdims=True)

    def row(c4, _):
        c = c4 * 4
        for u in range(4):
            for a in range(A):
                scr_ref[a * TN + c + u] = ewp_ref[idx_ref[0, 0,
                                                          (c + u) * A + a]]
        return ()

    jax.lax.fori_loop(0, TN // 4, row, ())
    out_ref[...] = functools.reduce(
        lambda x, y: x + y,
        [
            pltpu.bitcast(scr_ref[a * TN:(a + 1) * TN], jnp.bfloat16)
            .astype(jnp.float32) * attn3[:, a:a + 1, :1]
            for a in range(A)
        ],
    )


def _attention(anc2, gl, ga, mask3, w2t, ewp, c0, nc):
    return pl.pallas_call(
        _attn_kernel,
        grid=(nc // TN,),
        in_specs=[
            pl.BlockSpec((1, 1, A * TN), lambda i: (c0 + i, 0, 0),
                         memory_space=pltpu.SMEM),
            pl.BlockSpec((A * TN, H), lambda i: (c0 + i, 0)),
            pl.BlockSpec((A * TN, H), lambda i: (i, 0)),
            pl.BlockSpec((A * TN, 1), lambda i: (c0 + i, 0)),
            pl.BlockSpec((H, A), lambda i: (0, 0)),
            pl.BlockSpec((N_CODES * 4, PW, 128), lambda i: (0, 0, 0)),
        ],
        out_specs=pl.BlockSpec((TN, P4, 128), lambda i: (i, 0, 0)),
        out_shape=jax.ShapeDtypeStruct((nc, P4, 128), jnp.float32),
        scratch_shapes=[pltpu.VMEM((A * TN, PW, 128), jnp.int32)],
        compiler_params=pltpu.CompilerParams(
            dimension_semantics=("parallel",),
            vmem_limit_bytes=60 * 1024 * 1024,
        ),
    )(anc2, gl, ga, mask3, w2t, ewp)


def _pool_kernel(idx_ref, wgt_ref, cm_ref, c3_ref, bc_ref, out_ref):
    def row(r, _):
        acc = jnp.zeros((P4, 128), jnp.float32)
        cnt = jnp.float32(0.0)
        for m in range(48):
            acc = acc + wgt_ref[r, m] * c3_ref[idx_ref[r, m]]
            cnt = cnt + cm_ref[r, m]
        scale = 1.0 / jnp.maximum(jnp.full((P4, 128), cnt), 1.0)
        out_ref[r] = acc * scale + bc_ref[...]
        return ()

    jax.lax.fori_loop(0, TB, row, ())


def _pool(idx, wgt, cmf, c3, bc4):
    bv = idx.shape[0]
    return pl.pallas_call(
        _pool_kernel,
        grid=(bv // TB,),
        in_specs=[
            pl.BlockSpec((TB, 48), lambda i: (i, 0),
                         memory_space=pltpu.SMEM),
            pl.BlockSpec((TB, 48), lambda i: (i, 0),
                         memory_space=pltpu.SMEM),
            pl.BlockSpec((TB, 48), lambda i: (i, 0),
                         memory_space=pltpu.SMEM),
            pl.BlockSpec((N_CODES, P4, 128), lambda i: (0, 0, 0)),
            pl.BlockSpec((P4, 128), lambda i: (0, 0)),
        ],
        out_specs=pl.BlockSpec((TB, P4, 128), lambda i: (i, 0, 0)),
        out_shape=jax.ShapeDtypeStruct((bv, P4, 128), jnp.float32),
        compiler_params=pltpu.CompilerParams(
            dimension_semantics=("parallel",),
            vmem_limit_bytes=60 * 1024 * 1024,
        ),
    )(idx, wgt, cmf, c3, bc4)


def kernel(embed_table, w1, b1, w2, b2, wc, bc, masks, code_mask,
           leaves_list, ancestors_list, input_ids):
    del b2
    f32 = jnp.float32
    w_cat = jnp.concatenate([w1[:H, :], w1[H:, :], wc],
                            axis=1).astype(jnp.bfloat16)
    b_cat = jnp.concatenate(
        [jnp.zeros((H,), f32), b1, jnp.zeros((OUT,), f32)]).reshape(1, -1)
    ql, qa, ewp = _project(embed_table, w_cat, b_cat)

    gl = ql[leaves_list.reshape(-1).astype(jnp.int32)]
    anc_flat = ancestors_list.reshape(-1).astype(jnp.int32)
    anc2 = anc_flat.reshape(-1, 1, A * TN)
    madd = ((1.0 - masks) * VERY_NEG).reshape(N_CODES * A, 1)
    w2t = jnp.broadcast_to(w2, (H, A)).astype(jnp.bfloat16)
    nc = N_CODES // NCHUNK
    chunks = []
    for c in range(NCHUNK):
        ga = qa[anc_flat[c * nc * A:(c + 1) * nc * A]]
        chunks.append(
            _attention(anc2, gl, ga, madd, w2t, ewp, c * (nc // TN), nc))
    ctab = jnp.concatenate(chunks, axis=0)

    ids = input_ids.reshape(-1, 48).astype(jnp.int32)
    idx = jnp.maximum(ids - 1, 0)
    cmf = code_mask.reshape(-1, 48).astype(f32)
    wgt = cmf * (ids != 0).astype(f32)
    bc4 = bc.reshape(P4, 128)
    out = _pool(idx, wgt, cmf, ctab, bc4)
    B, V, _ = input_ids.shape
    return out.reshape(B, V, OUT)


# speedup vs baseline: 1.4226x; 1.0257x over previous
import functools

import jax
import jax.numpy as jnp
from jax.experimental import pallas as pl
from jax.experimental.pallas import tpu as pltpu

H = 768
A = 8
N_CODES = 10000
OUT = 512
VERY_NEG = -1e30

TM = 1000
TN = 200
NCHUNK = 2
TB = 64
P4 = OUT // 128
PW = OUT // 256


def _proj_kernel(e_ref, w_ref, b_ref, ql_ref, qa_ref, ewp_ref):
    eb = e_ref[...].astype(jnp.bfloat16)
    p = jnp.dot(eb, w_ref[...], preferred_element_type=jnp.float32)
    p = p + b_ref[...]
    ql_ref[...] = p[:, :H].astype(jnp.bfloat16)
    qa_ref[...] = p[:, H:2 * H].astype(jnp.bfloat16)
    pcs = []
    for r in range(PW):
        lo = p[:, 2 * H + r * 256:2 * H + r * 256 + 128]
        hi = p[:, 2 * H + r * 256 + 128:2 * H + r * 256 + 256]
        lob = pltpu.bitcast(
            lo.astype(jnp.bfloat16).astype(jnp.float32), jnp.uint32)
        hib = pltpu.bitcast(
            hi.astype(jnp.bfloat16).astype(jnp.float32), jnp.uint32)
        packed = (lob >> jnp.uint32(16)) | (hib & jnp.uint32(0xFFFF0000))
        pcs.append(pltpu.bitcast(packed, jnp.int32).reshape(TM, 1, 128))
    ewp_ref[...] = jnp.concatenate(pcs, axis=1)


def _project(embed_table, w, b):
    n, d = embed_table.shape[0], w.shape[1]
    return pl.pallas_call(
        _proj_kernel,
        grid=(n // TM,),
        in_specs=[
            pl.BlockSpec((TM, H), lambda i: (i, 0)),
            pl.BlockSpec((H, d), lambda i: (0, 0)),
            pl.BlockSpec((1, d), lambda i: (0, 0)),
        ],
        out_specs=[
            pl.BlockSpec((TM, H), lambda i: (i, 0)),
            pl.BlockSpec((TM, H), lambda i: (i, 0)),
            pl.BlockSpec((TM, PW, 128), lambda i: (i, 0, 0)),
        ],
        out_shape=[
            jax.ShapeDtypeStruct((n, H), jnp.bfloat16),
            jax.ShapeDtypeStruct((n, H), jnp.bfloat16),
            jax.ShapeDtypeStruct((n, PW, 128), jnp.int32),
        ],
        compiler_params=pltpu.CompilerParams(
            dimension_semantics=("parallel",),
            vmem_limit_bytes=100 * 1024 * 1024,
        ),
    )(embed_table, w, b)


def _attn_kernel(idx_ref, gl_ref, ga_ref, m_ref, w2t_ref, ewp_ref,
                 out_ref, scr_ref):
    h = jnp.maximum(gl_ref[...] + ga_ref[...], jnp.bfloat16(0.0))
    lg = jnp.dot(h, w2t_ref[...], preferred_element_type=jnp.float32)
    lg = lg + jnp.broadcast_to(m_ref[...], (A * TN, A))
    lg3 = lg.reshape(TN, A, A)
    mx = jnp.max(lg3, axis=1, keepdims=True)
    e3 = jnp.exp(lg3 - mx)
    attn3 = e3 / jnp.sum(e3, axis=1, keepdims=True)

    def row(c4, _):
        c = c4 * 4
        for u in range(4):
            for a in range(A):
                scr_ref[a * TN + c + u] = ewp_ref[idx_ref[0, 0,
                                                          (c + u) * A + a]]
        return ()

    jax.lax.fori_loop(0, TN // 4, row, ())
    out_ref[...] = functools.reduce(
        lambda x, y: x + y,
        [
            pltpu.bitcast(scr_ref[a * TN:(a + 1) * TN], jnp.bfloat16)
            .astype(jnp.float32) * attn3[:, a:a + 1, :1]
            for a in range(A)
        ],
    )


def _attention(anc2, gl, ga, mask3, w2t, ewp, c0, nc):
    return pl.pallas_call(
        _attn_kernel,
        grid=(nc // TN,),
        in_specs=[
            pl.BlockSpec((1, 1, A * TN), lambda i: (c0 + i, 0, 0),
                         memory_space=pltpu.SMEM),
            pl.BlockSpec((A * TN, H), lambda i: (c0 + i, 0)),
            pl.BlockSpec((A * TN, H), lambda i: (i, 0)),
            pl.BlockSpec((A * TN, 1), lambda i: (c0 + i, 0)),
            pl.BlockSpec((H, A), lambda i: (0, 0)),
            pl.BlockSpec((N_CODES * 4, PW, 128), lambda i: (0, 0, 0)),
        ],
        out_specs=pl.BlockSpec((TN, P4, 128), lambda i: (i, 0, 0)),
        out_shape=jax.ShapeDtypeStruct((nc, P4, 128), jnp.float32),
        scratch_shapes=[pltpu.VMEM((A * TN, PW, 128), jnp.int32)],
        compiler_params=pltpu.CompilerParams(
            dimension_semantics=("parallel",),
            vmem_limit_bytes=60 * 1024 * 1024,
        ),
    )(anc2, gl, ga, mask3, w2t, ewp)


def _pool_kernel(idx_ref, wgt_ref, cm_ref, c3_ref, bc_ref, out_ref):
    def row(r, _):
        acc = jnp.zeros((P4, 128), jnp.float32)
        cnt = jnp.float32(0.0)
        for m in range(48):
            acc = acc + wgt_ref[r, m] * c3_ref[idx_ref[r, m]]
            cnt = cnt + cm_ref[r, m]
        scale = 1.0 / jnp.maximum(jnp.full((P4, 128), cnt), 1.0)
        out_ref[r] = acc * scale + bc_ref[...]
        return ()

    jax.lax.fori_loop(0, TB, row, ())


def _pool(idx, wgt, cmf, c3, bc4):
    bv = idx.shape[0]
    return pl.pallas_call(
        _pool_kernel,
        grid=(bv // TB,),
        in_specs=[
            pl.BlockSpec((TB, 48), lambda i: (i, 0),
                         memory_space=pltpu.SMEM),
            pl.BlockSpec((TB, 48), lambda i: (i, 0),
                         memory_space=pltpu.SMEM),
            pl.BlockSpec((TB, 48), lambda i: (i, 0),
                         memory_space=pltpu.SMEM),
            pl.BlockSpec((N_CODES, P4, 128), lambda i: (0, 0, 0)),
            pl.BlockSpec((P4, 128), lambda i: (0, 0)),
        ],
        out_specs=pl.BlockSpec((TB, P4, 128), lambda i: (i, 0, 0)),
        out_shape=jax.ShapeDtypeStruct((bv, P4, 128), jnp.float32),
        compiler_params=pltpu.CompilerParams(
            dimension_semantics=("parallel",),
            vmem_limit_bytes=60 * 1024 * 1024,
        ),
    )(idx, wgt, cmf, c3, bc4)


def kernel(embed_table, w1, b1, w2, b2, wc, bc, masks, code_mask,
           leaves_list, ancestors_list, input_ids):
    del b2
    f32 = jnp.float32
    w_cat = jnp.concatenate([w1[:H, :], w1[H:, :], wc],
                            axis=1).astype(jnp.bfloat16)
    b_cat = jnp.concatenate(
        [jnp.zeros((H,), f32), b1, jnp.zeros((OUT,), f32)]).reshape(1, -1)
    ql, qa, ewp = _project(embed_table, w_cat, b_cat)

    gl = ql[leaves_list.reshape(-1).astype(jnp.int32)]
    anc_flat = ancestors_list.reshape(-1).astype(jnp.int32)
    anc2 = anc_flat.reshape(-1, 1, A * TN)
    madd = ((1.0 - masks) * VERY_NEG).reshape(N_CODES * A, 1)
    w2t = jnp.broadcast_to(w2, (H, A)).astype(jnp.bfloat16)
    nc = N_CODES // NCHUNK
    chunks = []
    for c in range(NCHUNK):
        ga = qa[anc_flat[c * nc * A:(c + 1) * nc * A]]
        chunks.append(
            _attention(anc2, gl, ga, madd, w2t, ewp, c * (nc // TN), nc))
    ctab = jnp.concatenate(chunks, axis=0)

    ids = input_ids.reshape(-1, 48).astype(jnp.int32)
    idx = jnp.maximum(ids - 1, 0)
    cmf = code_mask.reshape(-1, 48).astype(f32)
    wgt = cmf * (ids != 0).astype(f32)
    bc4 = bc.reshape(P4, 128)
    out = _pool(idx, wgt, cmf, ctab, bc4)
    B, V, _ = input_ids.shape
    return out.reshape(B, V, OUT)


# speedup vs baseline: 1.4251x; 1.0018x over previous
import functools

import jax
import jax.numpy as jnp
from jax.experimental import pallas as pl
from jax.experimental.pallas import tpu as pltpu

H = 768
A = 8
N_CODES = 10000
OUT = 512
VERY_NEG = -1e30

TM = 1000
TN = 200
NCHUNK = 1
TB = 64
P4 = OUT // 128
PW = OUT // 256


def _proj_kernel(e_ref, w_ref, b_ref, ql_ref, qa_ref, ewp_ref):
    eb = e_ref[...].astype(jnp.bfloat16)
    p = jnp.dot(eb, w_ref[...], preferred_element_type=jnp.float32)
    p = p + b_ref[...]
    ql_ref[...] = p[:, :H].astype(jnp.bfloat16)
    qa_ref[...] = p[:, H:2 * H].astype(jnp.bfloat16)
    pcs = []
    for r in range(PW):
        lo = p[:, 2 * H + r * 256:2 * H + r * 256 + 128]
        hi = p[:, 2 * H + r * 256 + 128:2 * H + r * 256 + 256]
        lob = pltpu.bitcast(
            lo.astype(jnp.bfloat16).astype(jnp.float32), jnp.uint32)
        hib = pltpu.bitcast(
            hi.astype(jnp.bfloat16).astype(jnp.float32), jnp.uint32)
        packed = (lob >> jnp.uint32(16)) | (hib & jnp.uint32(0xFFFF0000))
        pcs.append(pltpu.bitcast(packed, jnp.int32).reshape(TM, 1, 128))
    ewp_ref[...] = jnp.concatenate(pcs, axis=1)


def _project(embed_table, w, b):
    n, d = embed_table.shape[0], w.shape[1]
    return pl.pallas_call(
        _proj_kernel,
        grid=(n // TM,),
        in_specs=[
            pl.BlockSpec((TM, H), lambda i: (i, 0)),
            pl.BlockSpec((H, d), lambda i: (0, 0)),
            pl.BlockSpec((1, d), lambda i: (0, 0)),
        ],
        out_specs=[
            pl.BlockSpec((TM, H), lambda i: (i, 0)),
            pl.BlockSpec((TM, H), lambda i: (i, 0)),
            pl.BlockSpec((TM, PW, 128), lambda i: (i, 0, 0)),
        ],
        out_shape=[
            jax.ShapeDtypeStruct((n, H), jnp.bfloat16),
            jax.ShapeDtypeStruct((n, H), jnp.bfloat16),
            jax.ShapeDtypeStruct((n, PW, 128), jnp.int32),
        ],
        compiler_params=pltpu.CompilerParams(
            dimension_semantics=("parallel",),
            vmem_limit_bytes=100 * 1024 * 1024,
        ),
    )(embed_table, w, b)


def _attn_kernel(idx_ref, gl_ref, ga_ref, m_ref, w2t_ref, ewp_ref,
                 out_ref, scr_ref):
    h = jnp.maximum(gl_ref[...] + ga_ref[...], jnp.bfloat16(0.0))
    lg = jnp.dot(h, w2t_ref[...], preferred_element_type=jnp.float32)
    lg = lg + jnp.broadcast_to(m_ref[...], (A * TN, A))
    lg3 = lg.reshape(TN, A, A)
    mx = jnp.max(lg3, axis=1, keepdims=True)
    e3 = jnp.exp(lg3 - mx)
    attn3 = e3 / jnp.sum(e3, axis=1, keepdims=True)

    def row(c4, _):
        c = c4 * 4
        for u in range(4):
            for a in range(A):
                scr_ref[a * TN + c + u] = ewp_ref[idx_ref[0, 0,
                                                          (c + u) * A + a]]
        return ()

    jax.lax.fori_loop(0, TN // 4, row, ())
    out_ref[...] = functools.reduce(
        lambda x, y: x + y,
        [
            pltpu.bitcast(scr_ref[a * TN:(a + 1) * TN], jnp.bfloat16)
            .astype(jnp.float32) * attn3[:, a:a + 1, :1]
            for a in range(A)
        ],
    )


def _attention(anc2, gl, ga, mask3, w2t, ewp, c0, nc):
    return pl.pallas_call(
        _attn_kernel,
        grid=(nc // TN,),
        in_specs=[
            pl.BlockSpec((1, 1, A * TN), lambda i: (c0 + i, 0, 0),
                         memory_space=pltpu.SMEM),
            pl.BlockSpec((A * TN, H), lambda i: (c0 + i, 0)),
            pl.BlockSpec((A * TN, H), lambda i: (i, 0)),
            pl.BlockSpec((A * TN, 1), lambda i: (c0 + i, 0)),
            pl.BlockSpec((H, A), lambda i: (0, 0)),
            pl.BlockSpec((N_CODES * 4, PW, 128), lambda i: (0, 0, 0)),
        ],
        out_specs=pl.BlockSpec((TN, P4, 128), lambda i: (i, 0, 0)),
        out_shape=jax.ShapeDtypeStruct((nc, P4, 128), jnp.float32),
        scratch_shapes=[pltpu.VMEM((A * TN, PW, 128), jnp.int32)],
        compiler_params=pltpu.CompilerParams(
            dimension_semantics=("parallel",),
            vmem_limit_bytes=60 * 1024 * 1024,
        ),
    )(anc2, gl, ga, mask3, w2t, ewp)


def _pool_kernel(idx_ref, wgt_ref, cm_ref, c3_ref, bc_ref, out_ref):
    def row(r, _):
        acc = jnp.zeros((P4, 128), jnp.float32)
        cnt = jnp.float32(0.0)
        for m in range(48):
            acc = acc + wgt_ref[r, m] * c3_ref[idx_ref[r, m]]
            cnt = cnt + cm_ref[r, m]
        scale = 1.0 / jnp.maximum(jnp.full((P4, 128), cnt), 1.0)
        out_ref[r] = acc * scale + bc_ref[...]
        return ()

    jax.lax.fori_loop(0, TB, row, ())


def _pool(idx, wgt, cmf, c3, bc4):
    bv = idx.shape[0]
    return pl.pallas_call(
        _pool_kernel,
        grid=(bv // TB,),
        in_specs=[
            pl.BlockSpec((TB, 48), lambda i: (i, 0),
                         memory_space=pltpu.SMEM),
            pl.BlockSpec((TB, 48), lambda i: (i, 0),
                         memory_space=pltpu.SMEM),
            pl.BlockSpec((TB, 48), lambda i: (i, 0),
                         memory_space=pltpu.SMEM),
            pl.BlockSpec((N_CODES, P4, 128), lambda i: (0, 0, 0)),
            pl.BlockSpec((P4, 128), lambda i: (0, 0)),
        ],
        out_specs=pl.BlockSpec((TB, P4, 128), lambda i: (i, 0, 0)),
        out_shape=jax.ShapeDtypeStruct((bv, P4, 128), jnp.float32),
        compiler_params=pltpu.CompilerParams(
            dimension_semantics=("parallel",),
            vmem_limit_bytes=60 * 1024 * 1024,
        ),
    )(idx, wgt, cmf, c3, bc4)


def kernel(embed_table, w1, b1, w2, b2, wc, bc, masks, code_mask,
           leaves_list, ancestors_list, input_ids):
    del b2
    f32 = jnp.float32
    w_cat = jnp.concatenate([w1[:H, :], w1[H:, :], wc],
                            axis=1).astype(jnp.bfloat16)
    b_cat = jnp.concatenate(
        [jnp.zeros((H,), f32), b1, jnp.zeros((OUT,), f32)]).reshape(1, -1)
    ql, qa, ewp = _project(embed_table, w_cat, b_cat)

    gl = ql[leaves_list.reshape(-1).astype(jnp.int32)]
    anc_flat = ancestors_list.reshape(-1).astype(jnp.int32)
    anc2 = anc_flat.reshape(-1, 1, A * TN)
    madd = ((1.0 - masks) * VERY_NEG).reshape(N_CODES * A, 1)
    w2t = jnp.broadcast_to(w2, (H, A)).astype(jnp.bfloat16)
    nc = N_CODES // NCHUNK
    chunks = []
    for c in range(NCHUNK):
        ga = qa[anc_flat[c * nc * A:(c + 1) * nc * A]]
        chunks.append(
            _attention(anc2, gl, ga, madd, w2t, ewp, c * (nc // TN), nc))
    ctab = jnp.concatenate(chunks, axis=0)

    ids = input_ids.reshape(-1, 48).astype(jnp.int32)
    idx = jnp.maximum(ids - 1, 0)
    cmf = code_mask.reshape(-1, 48).astype(f32)
    wgt = cmf * (ids != 0).astype(f32)
    bc4 = bc.reshape(P4, 128)
    out = _pool(idx, wgt, cmf, ctab, bc4)
    B, V, _ = input_ids.shape
    return out.reshape(B, V, OUT)
